# baseline (speedup 1.0000x reference)
import functools

import jax
import jax.numpy as jnp
from jax import lax
from jax.experimental import pallas as pl
from jax.experimental.pallas import tpu as pltpu
from jax.experimental.pallas import tpu_sc as plsc

D = 128
L = 16
NC = 2
NS = 16
NW = NC * NS
NBUF = 4
FPAD = 128


def _sc_embed(cond_pad, cat_pad, cond_table, cat_table, cat_start_pad,
              f_real, pitch):
    BT = cond_pad.shape[0]
    ROWS = BT // NW
    NCOND = cond_table.shape[0]
    n_cs = cat_start_pad.shape[0]
    mesh = plsc.VectorSubcoreMesh(core_axis_name="c", subcore_axis_name="s")

    @functools.partial(
        pl.kernel,
        out_type=jax.ShapeDtypeStruct((BT * pitch, D), jnp.float32),
        mesh=mesh,
        compiler_params=pltpu.CompilerParams(needs_layout_passes=False),
        scratch_types=[
            pltpu.VMEM((n_cs,), jnp.int32),
            pltpu.VMEM((NCOND, D), jnp.float32),
            pltpu.VMEM((ROWS, FPAD), jnp.int32),
            pltpu.VMEM((ROWS, FPAD), jnp.int32),
            pltpu.VMEM((NBUF, FPAD, D), jnp.float32),
            pltpu.SemaphoreType.DMA,
            pltpu.SemaphoreType.DMA,
            pltpu.SemaphoreType.DMA,
            pltpu.SemaphoreType.DMA,
            pltpu.SemaphoreType.DMA,
            pltpu.SemaphoreType.DMA,
            pltpu.SemaphoreType.DMA,
            pltpu.SemaphoreType.DMA,
        ],
    )
    def k(cond_hbm, cat_hbm, condtab_hbm, cattab_hbm, cs_hbm, out_hbm,
          cs_v, ctab_v, cond_slab, full_slab, rcat,
          g0, g1, g2, g3, w0, w1, w2, w3):
        gsem = (g0, g1, g2, g3)
        wsem = (w0, w1, w2, w3)
        wid = lax.axis_index("s") * NC + lax.axis_index("c")
        base = wid * ROWS
        obase = wid * ROWS * pitch

        pltpu.sync_copy(cs_hbm, cs_v)
        pltpu.sync_copy(condtab_hbm, ctab_v)
        pltpu.sync_copy(cond_hbm.at[pl.ds(base, ROWS)], cond_slab)
        pltpu.sync_copy(cat_hbm.at[pl.ds(base, ROWS)], full_slab)

        @pl.loop(0, ROWS)
        def _(r):
            for kk in range(FPAD // L):
                sl = pl.ds(kk * L, L)
                starts = plsc.load_gather(cs_v, [cond_slab[r, sl]])
                full_slab[r, sl] = starts + full_slab[r, sl]

        def fire(c, b):
            pltpu.async_copy(cattab_hbm.at[full_slab.at[c]], rcat.at[b], gsem[b])

        def drain_write(b):
            pltpu.make_async_copy(
                rcat.at[b, pl.ds(0, pitch)],
                out_hbm.at[pl.ds(0, pitch)], wsem[b]).wait()

        def consume(c, b):
            pltpu.make_async_copy(
                cattab_hbm.at[pl.ds(0, FPAD)], rcat.at[b], gsem[b]).wait()

            @plsc.parallel_loop(0, f_real, unroll=2)
            def _(r):
                cid = plsc.load_gather(
                    cond_slab, [jnp.full((L,), c, jnp.int32),
                                jnp.full((L,), r, jnp.int32)])
                for kk in range(D // L):
                    colv = lax.iota(jnp.int32, L) + (kk * L)
                    vals = plsc.load_gather(ctab_v, [cid, colv])
                    plsc.addupdate(rcat.at[b, r, pl.ds(kk * L, L)], vals)

            pltpu.async_copy(rcat.at[b, pl.ds(0, pitch)],
                             out_hbm.at[pl.ds(obase + c * pitch, pitch)],
                             wsem[b])

        fire(0, 0)
        fire(1, 1)

        @pl.loop(0, ROWS, step=NBUF)
        def _(g):
            for b in range(NBUF):
                c = g + b
                nb = (b + 2) % NBUF

                @pl.when(c + 2 < ROWS)
                def _():
                    @pl.when(c >= 2)
                    def _():
                        drain_write(nb)

                    fire(c + 2, nb)

                consume(c, b)

        for b in range(NBUF):
            drain_write(b)

    return k(cond_pad, cat_pad, cond_table, cat_table, cat_start_pad)


def kernel(cond_ids, cat_ids, cond_table, cat_table, cat_start):
    bt, f = cond_ids.shape
    pitch = ((f + 7) // 8) * 8
    pad = FPAD - f
    cond_p = jnp.pad(cond_ids.astype(jnp.int32), ((0, 0), (0, pad)))
    cat_p = jnp.pad(cat_ids.astype(jnp.int32), ((0, 0), (0, pad)))
    cs = cat_start.astype(jnp.int32)
    n_pad = ((cs.shape[0] + 7) // 8) * 8
    cs_pad = jnp.zeros((n_pad,), jnp.int32).at[: cs.shape[0]].set(cs)
    out = _sc_embed(cond_p, cat_p, cond_table, cat_table, cs_pad, f, pitch)
    d = cond_table.shape[1]
    return out.reshape(bt, pitch, d)[:, :f, :]


# speedup vs baseline: 14.2234x; 14.2234x over previous
import functools

import jax
import jax.numpy as jnp
from jax import lax
from jax.experimental import pallas as pl
from jax.experimental.pallas import tpu as pltpu
from jax.experimental.pallas import tpu_sc as plsc

D = 128
L = 16
NC = 2
NS = 16
NW = NC * NS
NBUF = 4
FPAD = 128


def _sc_embed(cond_pad, cat_pad, cond_table, cat_table, cat_start_pad,
              f_real, pitch):
    BT = cond_pad.shape[0]
    ROWS = BT // NW
    NCOND = cond_table.shape[0]
    n_cs = cat_start_pad.shape[0]
    mesh = plsc.VectorSubcoreMesh(core_axis_name="c", subcore_axis_name="s")

    @functools.partial(
        pl.kernel,
        out_type=jax.ShapeDtypeStruct((BT * pitch, D), jnp.float32),
        mesh=mesh,
        compiler_params=pltpu.CompilerParams(needs_layout_passes=False),
        scratch_types=[
            pltpu.VMEM((n_cs,), jnp.int32),
            pltpu.VMEM((NCOND, D), jnp.float32),
            pltpu.VMEM((ROWS, FPAD), jnp.int32),
            pltpu.VMEM((ROWS, FPAD), jnp.int32),
            pltpu.VMEM((NBUF, FPAD, D), jnp.float32),
            pltpu.SemaphoreType.DMA,
            pltpu.SemaphoreType.DMA,
            pltpu.SemaphoreType.DMA,
            pltpu.SemaphoreType.DMA,
            pltpu.SemaphoreType.DMA,
            pltpu.SemaphoreType.DMA,
            pltpu.SemaphoreType.DMA,
            pltpu.SemaphoreType.DMA,
        ],
    )
    def k(cond_hbm, cat_hbm, condtab_hbm, cattab_hbm, cs_hbm, out_hbm,
          cs_v, ctab_v, cond_slab, full_slab, rcat,
          g0, g1, g2, g3, w0, w1, w2, w3):
        gsem = (g0, g1, g2, g3)
        wsem = (w0, w1, w2, w3)
        wid = lax.axis_index("s") * NC + lax.axis_index("c")
        base = wid * ROWS
        obase = wid * ROWS * pitch

        pltpu.sync_copy(cs_hbm, cs_v)
        pltpu.sync_copy(condtab_hbm, ctab_v)
        pltpu.sync_copy(cond_hbm.at[pl.ds(base, ROWS)], cond_slab)
        pltpu.sync_copy(cat_hbm.at[pl.ds(base, ROWS)], full_slab)

        ncat = cattab_hbm.shape[0]
        @pl.loop(0, ROWS)
        def _(r):
            for kk in range(FPAD // L):
                sl = pl.ds(kk * L, L)
                starts = plsc.load_gather(cs_v, [cond_slab[r, sl]])
                real = starts + full_slab[r, sl]
                if (kk + 1) * L <= f_real:
                    full_slab[r, sl] = real
                else:
                    colf = lax.iota(jnp.int32, L) + (kk * L)
                    spread = ((base + r) * FPAD + colf) % ncat
                    full_slab[r, sl] = jnp.where(colf < f_real, real, spread)

        def fire(c, b):
            pltpu.async_copy(cattab_hbm.at[full_slab.at[c]], rcat.at[b], gsem[b])

        def drain_write(b):
            pltpu.make_async_copy(
                rcat.at[b, pl.ds(0, pitch)],
                out_hbm.at[pl.ds(0, pitch)], wsem[b]).wait()

        def consume(c, b):
            pltpu.make_async_copy(
                cattab_hbm.at[pl.ds(0, FPAD)], rcat.at[b], gsem[b]).wait()

            @plsc.parallel_loop(0, f_real, unroll=2)
            def _(r):
                cid = plsc.load_gather(
                    cond_slab, [jnp.full((L,), c, jnp.int32),
                                jnp.full((L,), r, jnp.int32)])
                for kk in range(D // L):
                    colv = lax.iota(jnp.int32, L) + (kk * L)
                    vals = plsc.load_gather(ctab_v, [cid, colv])
                    plsc.addupdate(rcat.at[b, r, pl.ds(kk * L, L)], vals)

            pltpu.async_copy(rcat.at[b, pl.ds(0, pitch)],
                             out_hbm.at[pl.ds(obase + c * pitch, pitch)],
                             wsem[b])

        fire(0, 0)
        fire(1, 1)

        @pl.loop(0, ROWS, step=NBUF)
        def _(g):
            for b in range(NBUF):
                c = g + b
                nb = (b + 2) % NBUF

                @pl.when(c + 2 < ROWS)
                def _():
                    @pl.when(c >= 2)
                    def _():
                        drain_write(nb)

                    fire(c + 2, nb)

                consume(c, b)

        for b in range(NBUF):
            drain_write(b)

    return k(cond_pad, cat_pad, cond_table, cat_table, cat_start_pad)


def kernel(cond_ids, cat_ids, cond_table, cat_table, cat_start):
    bt, f = cond_ids.shape
    pitch = ((f + 7) // 8) * 8
    pad = FPAD - f
    cond_p = jnp.pad(cond_ids.astype(jnp.int32), ((0, 0), (0, pad)))
    cat_p = jnp.pad(cat_ids.astype(jnp.int32), ((0, 0), (0, pad)))
    cs = cat_start.astype(jnp.int32)
    n_pad = ((cs.shape[0] + 7) // 8) * 8
    cs_pad = jnp.zeros((n_pad,), jnp.int32).at[: cs.shape[0]].set(cs)
    out = _sc_embed(cond_p, cat_p, cond_table, cat_table, cs_pad, f, pitch)
    d = cond_table.shape[1]
    return out.reshape(bt, pitch, d)[:, :f, :]


# speedup vs baseline: 15.0673x; 1.0593x over previous
import functools

import jax
import jax.numpy as jnp
from jax import lax
from jax.experimental import pallas as pl
from jax.experimental.pallas import tpu as pltpu
from jax.experimental.pallas import tpu_sc as plsc

D = 128
L = 16
NC = 2
NS = 16
NW = NC * NS
NBUF = 4
FPAD = 128


def _sc_embed(cond_pad, cat_pad, cond_table, cat_table, cat_start_pad,
              f_real, pitch):
    BT = cond_pad.shape[0]
    ROWS = BT // NW
    NCOND = cond_table.shape[0]
    n_cs = cat_start_pad.shape[0]
    mesh = plsc.VectorSubcoreMesh(core_axis_name="c", subcore_axis_name="s")

    @functools.partial(
        pl.kernel,
        out_type=jax.ShapeDtypeStruct((BT, f_real, D), jnp.float32),
        mesh=mesh,
        compiler_params=pltpu.CompilerParams(needs_layout_passes=False),
        scratch_types=[
            pltpu.VMEM((n_cs,), jnp.int32),
            pltpu.VMEM((NCOND, D), jnp.float32),
            pltpu.VMEM((ROWS, FPAD), jnp.int32),
            pltpu.VMEM((ROWS, FPAD), jnp.int32),
            pltpu.VMEM((NBUF, FPAD, D), jnp.float32),
            pltpu.SemaphoreType.DMA,
            pltpu.SemaphoreType.DMA,
            pltpu.SemaphoreType.DMA,
            pltpu.SemaphoreType.DMA,
            pltpu.SemaphoreType.DMA,
            pltpu.SemaphoreType.DMA,
            pltpu.SemaphoreType.DMA,
            pltpu.SemaphoreType.DMA,
        ],
    )
    def k(cond_hbm, cat_hbm, condtab_hbm, cattab_hbm, cs_hbm, out_hbm,
          cs_v, ctab_v, cond_slab, full_slab, rcat,
          g0, g1, g2, g3, w0, w1, w2, w3):
        gsem = (g0, g1, g2, g3)
        wsem = (w0, w1, w2, w3)
        wid = lax.axis_index("s") * NC + lax.axis_index("c")
        base = wid * ROWS
        obase = wid * ROWS * pitch

        pltpu.sync_copy(cs_hbm, cs_v)
        pltpu.sync_copy(condtab_hbm, ctab_v)
        pltpu.sync_copy(cond_hbm.at[pl.ds(base, ROWS)], cond_slab)
        pltpu.sync_copy(cat_hbm.at[pl.ds(base, ROWS)], full_slab)

        ncat = cattab_hbm.shape[0]
        @pl.loop(0, ROWS)
        def _(r):
            for kk in range(FPAD // L):
                sl = pl.ds(kk * L, L)
                starts = plsc.load_gather(cs_v, [cond_slab[r, sl]])
                real = starts + full_slab[r, sl]
                if (kk + 1) * L <= f_real:
                    full_slab[r, sl] = real
                else:
                    colf = lax.iota(jnp.int32, L) + (kk * L)
                    spread = ((base + r) * FPAD + colf) % ncat
                    full_slab[r, sl] = jnp.where(colf < f_real, real, spread)

        def fire(c, b):
            pltpu.async_copy(cattab_hbm.at[full_slab.at[c]], rcat.at[b], gsem[b])

        def drain_write(b):
            pltpu.make_async_copy(
                rcat.at[b, pl.ds(0, f_real)],
                out_hbm.at[0], wsem[b]).wait()

        def consume(c, b):
            pltpu.make_async_copy(
                cattab_hbm.at[pl.ds(0, FPAD)], rcat.at[b], gsem[b]).wait()

            @plsc.parallel_loop(0, f_real, unroll=2)
            def _(r):
                cid = plsc.load_gather(
                    cond_slab, [jnp.full((L,), c, jnp.int32),
                                jnp.full((L,), r, jnp.int32)])
                for kk in range(D // L):
                    colv = lax.iota(jnp.int32, L) + (kk * L)
                    vals = plsc.load_gather(ctab_v, [cid, colv])
                    plsc.addupdate(rcat.at[b, r, pl.ds(kk * L, L)], vals)

            pltpu.async_copy(rcat.at[b, pl.ds(0, f_real)],
                             out_hbm.at[base + c],
                             wsem[b])

        fire(0, 0)
        fire(1, 1)

        @pl.loop(0, ROWS, step=NBUF)
        def _(g):
            for b in range(NBUF):
                c = g + b
                nb = (b + 2) % NBUF

                @pl.when(c + 2 < ROWS)
                def _():
                    @pl.when(c >= 2)
                    def _():
                        drain_write(nb)

                    fire(c + 2, nb)

                consume(c, b)

        for b in range(NBUF):
            drain_write(b)

    return k(cond_pad, cat_pad, cond_table, cat_table, cat_start_pad)


def kernel(cond_ids, cat_ids, cond_table, cat_table, cat_start):
    bt, f = cond_ids.shape
    pitch = ((f + 7) // 8) * 8
    pad = FPAD - f
    cond_p = jnp.pad(cond_ids.astype(jnp.int32), ((0, 0), (0, pad)))
    cat_p = jnp.pad(cat_ids.astype(jnp.int32), ((0, 0), (0, pad)))
    cs = cat_start.astype(jnp.int32)
    n_pad = ((cs.shape[0] + 7) // 8) * 8
    cs_pad = jnp.zeros((n_pad,), jnp.int32).at[: cs.shape[0]].set(cs)
    return _sc_embed(cond_p, cat_p, cond_table, cat_table, cs_pad, f, pitch)
